# baseline (speedup 1.0000x reference)
import functools

import jax
import jax.numpy as jnp
from jax.experimental import pallas as pl

NUM_CODES = 8192
CODE_DIM = 32
BETA = 0.25
BLK_M = 256
N_CHUNKS = 4
CHUNK = NUM_CODES // N_CHUNKS


def _rnd_bf16(x):
    return x.astype(jnp.bfloat16).astype(jnp.float32)


def _vq_block_kernel(z_ref, zsq_ref, cb_ref, esq_ref,
                     zq_ref, idx_ref, loss_ref):
    i = pl.program_id(0)
    z = z_ref[...]
    s = jax.lax.dot_general(z.astype(jnp.bfloat16),
                            cb_ref[...].astype(jnp.bfloat16),
                            (((1,), (1,)), ((), ())),
                            preferred_element_type=jnp.float32)
    d = (zsq_ref[...] + esq_ref[...]) - 2.0 * s

    acc_v = None
    acc_i = None
    iota = jax.lax.broadcasted_iota(jnp.int32, (BLK_M, CHUNK), 1)
    for c in range(N_CHUNKS):
        dc = d[:, c * CHUNK:(c + 1) * CHUNK]
        mc = jnp.min(dc, axis=1, keepdims=True)
        ic = jnp.min(jnp.where(dc == mc, iota, NUM_CODES),
                     axis=1, keepdims=True) + c * CHUNK
        if c == 0:
            acc_v, acc_i = _rnd_bf16(mc), ic
        else:
            lt = mc < acc_v
            acc_v = jnp.where(lt, _rnd_bf16(mc), acc_v)
            acc_i = jnp.where(lt, ic, acc_i)

    iota_n = jax.lax.broadcasted_iota(jnp.int32, (BLK_M, NUM_CODES), 1)
    onehot = (iota_n == acc_i).astype(jnp.float32)
    zq = jnp.dot(onehot, cb_ref[...],
                 preferred_element_type=jnp.float32,
                 precision=jax.lax.Precision.HIGHEST)
    zq_ref[...] = zq
    idx_ref[...] = acc_i
    diff = zq - z
    part = jnp.sum(diff * diff).reshape(1, 1)

    @pl.when(i == 0)
    def _():
        loss_ref[...] = part

    @pl.when(i > 0)
    def _():
        loss_ref[...] += part


def kernel(z_e, codebook):
    B, N, C = z_e.shape
    z_flat = z_e.reshape(-1, C)
    R = z_flat.shape[0]
    n_blocks = R // BLK_M
    z_sq = jnp.sum(z_e ** 2, axis=2).reshape(R, 1)
    e_sq = jnp.sum(codebook ** 2, axis=1)[None, :]

    zq, idx, loss = pl.pallas_call(
        _vq_block_kernel,
        grid=(n_blocks,),
        in_specs=[
            pl.BlockSpec((BLK_M, C), lambda i: (i, 0)),
            pl.BlockSpec((BLK_M, 1), lambda i: (i, 0)),
            pl.BlockSpec((NUM_CODES, C), lambda i: (0, 0)),
            pl.BlockSpec((1, NUM_CODES), lambda i: (0, 0)),
        ],
        out_specs=[
            pl.BlockSpec((BLK_M, C), lambda i: (i, 0)),
            pl.BlockSpec((BLK_M, 1), lambda i: (i, 0)),
            pl.BlockSpec((1, 1), lambda i: (0, 0)),
        ],
        out_shape=[
            jax.ShapeDtypeStruct((R, C), jnp.float32),
            jax.ShapeDtypeStruct((R, 1), jnp.int32),
            jax.ShapeDtypeStruct((1, 1), jnp.float32),
        ],
    )(z_flat, z_sq, codebook, e_sq)

    z_q = zq.reshape(B, N, C)
    indices = idx.reshape(B, N)
    mse = loss[0, 0] / (R * C)
    vq_loss = (1.0 + BETA) * mse / C
    return (z_q, indices, vq_loss)


# speedup vs baseline: 1.9892x; 1.9892x over previous
import jax
import jax.numpy as jnp
from jax.experimental import pallas as pl
from jax.experimental.pallas import tpu as pltpu

NUM_CODES = 8192
CODE_DIM = 32
BETA = 0.25
BLK_M = 256
N_CHUNKS = 4
CHUNK = NUM_CODES // N_CHUNKS


def _rnd_bf16(x):
    return x.astype(jnp.bfloat16).astype(jnp.float32)


def _vq_block_kernel(z16_ref, zsq_ref, cb16_ref, cblo_ref, esq_ref,
                     zq_ref, idx_ref, loss_ref):
    i = pl.program_id(0)
    s = jax.lax.dot_general(z16_ref[...], cb16_ref[...],
                            (((1,), (1,)), ((), ())),
                            preferred_element_type=jnp.float32)
    d = (zsq_ref[...] + esq_ref[...]) - 2.0 * s

    acc_v = acc_m = acc_i = None
    iota = jax.lax.broadcasted_iota(jnp.int32, (BLK_M, CHUNK), 1)
    for c in range(N_CHUNKS):
        dc = d[:, c * CHUNK:(c + 1) * CHUNK]
        mc = jnp.min(dc, axis=1, keepdims=True)
        ic = jnp.min(jnp.where(dc == mc, iota, NUM_CODES),
                     axis=1, keepdims=True) + c * CHUNK
        if c == 0:
            acc_v, acc_m, acc_i = _rnd_bf16(mc), mc, ic
        else:
            lt = mc < acc_v
            acc_v = jnp.where(lt, _rnd_bf16(mc), acc_v)
            acc_m = jnp.where(lt, mc, acc_m)
            acc_i = jnp.where(lt, ic, acc_i)

    iota_n = jax.lax.broadcasted_iota(jnp.int32, (BLK_M, NUM_CODES), 1)
    onehot = (iota_n == acc_i).astype(jnp.bfloat16)
    zq = (jnp.dot(onehot, cb16_ref[...], preferred_element_type=jnp.float32)
          + jnp.dot(onehot, cblo_ref[...], preferred_element_type=jnp.float32))
    zq_ref[...] = zq
    idx_ref[...] = acc_i
    part = jnp.sum(acc_m).reshape(1, 1)

    @pl.when(i == 0)
    def _():
        loss_ref[...] = part

    @pl.when(i > 0)
    def _():
        loss_ref[...] += part


def kernel(z_e, codebook):
    B, N, C = z_e.shape
    z_flat = z_e.reshape(-1, C)
    R = z_flat.shape[0]
    n_blocks = R // BLK_M
    z_sq = jnp.sum(z_e ** 2, axis=2).reshape(R, 1)
    e_sq = jnp.sum(codebook ** 2, axis=1)[None, :]
    z16 = z_flat.astype(jnp.bfloat16)
    cb16 = codebook.astype(jnp.bfloat16)
    cb_lo = (codebook - cb16.astype(jnp.float32)).astype(jnp.bfloat16)

    zq, idx, loss = pl.pallas_call(
        _vq_block_kernel,
        grid=(n_blocks,),
        in_specs=[
            pl.BlockSpec((BLK_M, C), lambda i: (i, 0)),
            pl.BlockSpec((BLK_M, 1), lambda i: (i, 0)),
            pl.BlockSpec((NUM_CODES, C), lambda i: (0, 0)),
            pl.BlockSpec((NUM_CODES, C), lambda i: (0, 0)),
            pl.BlockSpec((1, NUM_CODES), lambda i: (0, 0)),
        ],
        out_specs=[
            pl.BlockSpec((BLK_M, C), lambda i: (i, 0)),
            pl.BlockSpec((BLK_M, 1), lambda i: (i, 0)),
            pl.BlockSpec((1, 1), lambda i: (0, 0)),
        ],
        out_shape=[
            jax.ShapeDtypeStruct((R, C), jnp.float32),
            jax.ShapeDtypeStruct((R, 1), jnp.int32),
            jax.ShapeDtypeStruct((1, 1), jnp.float32),
        ],
        compiler_params=pltpu.CompilerParams(
            dimension_semantics=("arbitrary",)),
    )(z16, z_sq, cb16, cb_lo, e_sq)

    z_q = zq.reshape(B, N, C)
    indices = idx.reshape(B, N)
    mse = loss[0, 0] / (R * C)
    vq_loss = (1.0 + BETA) * mse / C
    return (z_q, indices, vq_loss)


# speedup vs baseline: 2.3405x; 1.1766x over previous
import jax
import jax.numpy as jnp
from jax.experimental import pallas as pl
from jax.experimental.pallas import tpu as pltpu

NUM_CODES = 8192
CODE_DIM = 32
BETA = 0.25
BLK_M = 256
N_CHUNKS = 4
CHUNK = NUM_CODES // N_CHUNKS


def _rnd_bf16(x):
    return x.astype(jnp.bfloat16).astype(jnp.float32)


def _vq_block_kernel(z16_ref, zsq_ref, cb16_ref, cbcat_ref, esq_ref,
                     zq_ref, idx_ref, loss_ref):
    s = jax.lax.dot_general(z16_ref[...], cb16_ref[...],
                            (((1,), (1,)), ((), ())),
                            preferred_element_type=jnp.float32)
    d = (zsq_ref[...] + esq_ref[...]) - 2.0 * s

    acc_v = acc_m = acc_i = None
    iota = jax.lax.broadcasted_iota(jnp.int32, (BLK_M, CHUNK), 1)
    for c in range(N_CHUNKS):
        dc = d[:, c * CHUNK:(c + 1) * CHUNK]
        mc = jnp.min(dc, axis=1, keepdims=True)
        ic = jnp.min(jnp.where(dc == mc, iota, NUM_CODES),
                     axis=1, keepdims=True) + c * CHUNK
        if c == 0:
            acc_v, acc_m, acc_i = _rnd_bf16(mc), mc, ic
        else:
            lt = mc < acc_v
            acc_v = jnp.where(lt, _rnd_bf16(mc), acc_v)
            acc_m = jnp.where(lt, mc, acc_m)
            acc_i = jnp.where(lt, ic, acc_i)

    iota_n = jax.lax.broadcasted_iota(jnp.int32, (BLK_M, NUM_CODES), 1)
    onehot = (iota_n == acc_i).astype(jnp.bfloat16)
    zq2 = jnp.dot(onehot, cbcat_ref[...],
                  preferred_element_type=jnp.float32)
    zq_ref[...] = zq2[:, :CODE_DIM] + zq2[:, CODE_DIM:]
    idx_ref[...] = acc_i
    loss_ref[...] = jnp.sum(acc_m).reshape(1, 1, 1)


def kernel(z_e, codebook):
    B, N, C = z_e.shape
    z_flat = z_e.reshape(-1, C)
    R = z_flat.shape[0]
    n_blocks = R // BLK_M
    z_sq = jnp.sum(z_e ** 2, axis=2).reshape(R, 1)
    e_sq = jnp.sum(codebook ** 2, axis=1)[None, :]
    z16 = z_flat.astype(jnp.bfloat16)
    cb16 = codebook.astype(jnp.bfloat16)
    cb_lo = (codebook - cb16.astype(jnp.float32)).astype(jnp.bfloat16)
    cbcat = jnp.concatenate([cb16, cb_lo], axis=1)

    zq, idx, loss = pl.pallas_call(
        _vq_block_kernel,
        grid=(n_blocks,),
        in_specs=[
            pl.BlockSpec((BLK_M, C), lambda i: (i, 0)),
            pl.BlockSpec((BLK_M, 1), lambda i: (i, 0)),
            pl.BlockSpec((NUM_CODES, C), lambda i: (0, 0)),
            pl.BlockSpec((NUM_CODES, 2 * C), lambda i: (0, 0)),
            pl.BlockSpec((1, NUM_CODES), lambda i: (0, 0)),
        ],
        out_specs=[
            pl.BlockSpec((BLK_M, C), lambda i: (i, 0)),
            pl.BlockSpec((BLK_M, 1), lambda i: (i, 0)),
            pl.BlockSpec((1, 1, 1), lambda i: (i, 0, 0)),
        ],
        out_shape=[
            jax.ShapeDtypeStruct((R, C), jnp.float32),
            jax.ShapeDtypeStruct((R, 1), jnp.int32),
            jax.ShapeDtypeStruct((n_blocks, 1, 1), jnp.float32),
        ],
        compiler_params=pltpu.CompilerParams(
            dimension_semantics=("parallel",)),
    )(z16, z_sq, cb16, cbcat, e_sq)

    z_q = zq.reshape(B, N, C)
    indices = idx.reshape(B, N)
    mse = jnp.sum(loss) / (R * C)
    vq_loss = (1.0 + BETA) * mse / C
    return (z_q, indices, vq_loss)


# speedup vs baseline: 2.5174x; 1.0756x over previous
import jax
import jax.numpy as jnp
from jax.experimental import pallas as pl
from jax.experimental.pallas import tpu as pltpu

NUM_CODES = 8192
CODE_DIM = 32
BETA = 0.25
BLK_M = 256
N_CHUNKS = 4
CHUNK = NUM_CODES // N_CHUNKS


def _rnd_bf16(x):
    return x.astype(jnp.bfloat16).astype(jnp.float32)


def _vq_block_kernel(z16_ref, zsq_ref, cb16_ref, cbcat_ref, esq_ref,
                     zq_ref, idx_ref, loss_ref):
    s = jax.lax.dot_general(z16_ref[...], cb16_ref[...],
                            (((1,), (1,)), ((), ())),
                            preferred_element_type=jnp.float32)
    d = (zsq_ref[...] + esq_ref[...]) - 2.0 * s

    acc_v = acc_m = acc_i = None
    SEG = 128
    NSEG = CHUNK // SEG
    iota_seg = jax.lax.broadcasted_iota(jnp.int32, (BLK_M, SEG), 1)
    for c in range(N_CHUNKS):
        base = c * CHUNK
        v = d[:, base:base + SEG]
        seg_no = jnp.zeros((BLK_M, SEG), jnp.int32)
        for s in range(1, NSEG):
            vs = d[:, base + s * SEG:base + (s + 1) * SEG]
            lt = vs < v
            v = jnp.where(lt, vs, v)
            seg_no = jnp.where(lt, s, seg_no)
        ixg = seg_no * SEG + iota_seg
        mc = jnp.min(v, axis=1, keepdims=True)
        ic = jnp.min(jnp.where(v == mc, ixg, NUM_CODES),
                     axis=1, keepdims=True) + base
        if c == 0:
            acc_v, acc_m, acc_i = _rnd_bf16(mc), mc, ic
        else:
            lt = mc < acc_v
            acc_v = jnp.where(lt, _rnd_bf16(mc), acc_v)
            acc_m = jnp.where(lt, mc, acc_m)
            acc_i = jnp.where(lt, ic, acc_i)

    iota_n = jax.lax.broadcasted_iota(jnp.int32, (BLK_M, NUM_CODES), 1)
    onehot = (iota_n == acc_i).astype(jnp.bfloat16)
    zq2 = jnp.dot(onehot, cbcat_ref[...],
                  preferred_element_type=jnp.float32)
    zq_ref[...] = zq2[:, :CODE_DIM] + zq2[:, CODE_DIM:]
    idx_ref[...] = acc_i
    loss_ref[...] = jnp.sum(acc_m).reshape(1, 1, 1)


def kernel(z_e, codebook):
    B, N, C = z_e.shape
    z_flat = z_e.reshape(-1, C)
    R = z_flat.shape[0]
    n_blocks = R // BLK_M
    z_sq = jnp.sum(z_e ** 2, axis=2).reshape(R, 1)
    e_sq = jnp.sum(codebook ** 2, axis=1)[None, :]
    z16 = z_flat.astype(jnp.bfloat16)
    cb16 = codebook.astype(jnp.bfloat16)
    cb_lo = (codebook - cb16.astype(jnp.float32)).astype(jnp.bfloat16)
    cbcat = jnp.concatenate([cb16, cb_lo], axis=1)

    zq, idx, loss = pl.pallas_call(
        _vq_block_kernel,
        grid=(n_blocks,),
        in_specs=[
            pl.BlockSpec((BLK_M, C), lambda i: (i, 0)),
            pl.BlockSpec((BLK_M, 1), lambda i: (i, 0)),
            pl.BlockSpec((NUM_CODES, C), lambda i: (0, 0)),
            pl.BlockSpec((NUM_CODES, 2 * C), lambda i: (0, 0)),
            pl.BlockSpec((1, NUM_CODES), lambda i: (0, 0)),
        ],
        out_specs=[
            pl.BlockSpec((BLK_M, C), lambda i: (i, 0)),
            pl.BlockSpec((BLK_M, 1), lambda i: (i, 0)),
            pl.BlockSpec((1, 1, 1), lambda i: (i, 0, 0)),
        ],
        out_shape=[
            jax.ShapeDtypeStruct((R, C), jnp.float32),
            jax.ShapeDtypeStruct((R, 1), jnp.int32),
            jax.ShapeDtypeStruct((n_blocks, 1, 1), jnp.float32),
        ],
        compiler_params=pltpu.CompilerParams(
            dimension_semantics=("parallel",)),
    )(z16, z_sq, cb16, cbcat, e_sq)

    z_q = zq.reshape(B, N, C)
    indices = idx.reshape(B, N)
    mse = jnp.sum(loss) / (R * C)
    vq_loss = (1.0 + BETA) * mse / C
    return (z_q, indices, vq_loss)
